# baseline (device time: 48637 ns/iter reference)
import os

import jax
import jax.numpy as jnp
from jax import lax
from jax.experimental import pallas as pl
from jax.experimental.pallas import tpu as pltpu

N_DEV = 16

_CAST = os.environ.get("K_CAST", "fp8")
_A2A = os.environ.get("K_A2A", "1") == "1"
_GEMM = os.environ.get("K_GEMM", "1") == "1"

_DeviceIdType = getattr(pl, "DeviceIdType", None) or pltpu.DeviceIdType


def kernel(x, w_mat, scale_x, scale_w):
    m_per, k = x.shape
    _, n = w_mat.shape
    n_per = n // N_DEV
    m_out = m_per * N_DEV

    def body(x_ref, w_ref, sx_ref, sw_ref, out_ref, y_ref, stage_ref,
             send_sems, recv_sems):
        my = lax.axis_index("i")

        scale = sx_ref[0] * sw_ref[0]
        if _GEMM:
            xv = x_ref[...]
            wv = w_ref[...]
            if _CAST == "fp8" and xv.dtype == jnp.float32:
                xv = xv.astype(jnp.float8_e4m3fn)
                wv = wv.astype(jnp.float8_e4m3fn)
            elif _CAST == "bf16" and xv.dtype == jnp.float32:
                xv = xv.astype(jnp.bfloat16)
                wv = wv.astype(jnp.bfloat16)
            acc = lax.dot_general(
                xv, wv, (((1,), (0,)), ((), ())),
                preferred_element_type=jnp.float32,
                precision=lax.Precision.DEFAULT,
            ) * scale
            y_ref[...] = acc
        else:
            y_ref[...] = jnp.zeros((m_per, n), jnp.float32)

        out_ref[pl.ds(my * m_per, m_per), :] = y_ref[:, pl.ds(my * n_per, n_per)]
        for s in range(1, N_DEV):
            dst = lax.rem(my + s, N_DEV)
            stage_ref[s] = y_ref[:, pl.ds(dst * n_per, n_per)]

        if _A2A:
            rdmas = []
            for s in range(1, N_DEV):
                dst = lax.rem(my + s, N_DEV)
                rdma = pltpu.make_async_remote_copy(
                    src_ref=stage_ref.at[s],
                    dst_ref=out_ref.at[pl.ds(my * m_per, m_per), :],
                    send_sem=send_sems.at[s],
                    recv_sem=recv_sems.at[s],
                    device_id=(dst,),
                    device_id_type=_DeviceIdType.MESH,
                )
                rdma.start()
                rdmas.append(rdma)
            for rdma in rdmas:
                rdma.wait()


    return pl.pallas_call(
        body,
        out_shape=jax.ShapeDtypeStruct((m_out, n_per), jnp.float32),
        in_specs=[
            pl.BlockSpec(memory_space=pltpu.VMEM),
            pl.BlockSpec(memory_space=pltpu.VMEM),
            pl.BlockSpec(memory_space=pltpu.SMEM),
            pl.BlockSpec(memory_space=pltpu.SMEM),
        ],
        out_specs=pl.BlockSpec(memory_space=pltpu.VMEM),
        scratch_shapes=[
            pltpu.VMEM((m_per, n), jnp.float32),
            pltpu.VMEM((N_DEV, m_per, n_per), jnp.float32),
            pltpu.SemaphoreType.DMA((N_DEV,)),
            pltpu.SemaphoreType.DMA((N_DEV,)),
        ],
        compiler_params=pltpu.CompilerParams(
            vmem_limit_bytes=100 * 1024 * 1024,
        ),
    )(x, w_mat, scale_x, scale_w)


# device time: 38698 ns/iter; 1.2568x vs baseline; 1.2568x over previous
import os

import jax
import jax.numpy as jnp
from jax import lax
from jax.experimental import pallas as pl
from jax.experimental.pallas import tpu as pltpu

N_DEV = 16

_CAST = os.environ.get("K_CAST", "fp8")
_A2A = os.environ.get("K_A2A", "1") == "1"
_GEMM = os.environ.get("K_GEMM", "1") == "1"

_DeviceIdType = getattr(pl, "DeviceIdType", None) or pltpu.DeviceIdType


def kernel(x, w_mat, scale_x, scale_w):
    m_per, k = x.shape
    _, n = w_mat.shape
    n_per = n // N_DEV
    m_out = m_per * N_DEV

    def body(x_ref, w_ref, sx_ref, sw_ref, out_ref, y_ref, stage_ref,
             rstage_ref, send_sems, recv_sems):
        my = lax.axis_index("i")

        scale = sx_ref[0] * sw_ref[0]
        if _GEMM:
            xv = x_ref[...]
            wv = w_ref[...]
            if _CAST == "fp8" and xv.dtype == jnp.float32:
                xv = xv.astype(jnp.float8_e4m3fn)
                wv = wv.astype(jnp.float8_e4m3fn)
            elif _CAST == "bf16" and xv.dtype == jnp.float32:
                xv = xv.astype(jnp.bfloat16)
                wv = wv.astype(jnp.bfloat16)
            acc = lax.dot_general(
                xv, wv, (((1,), (0,)), ((), ())),
                preferred_element_type=jnp.float32,
                precision=lax.Precision.DEFAULT,
            ) * scale
            y_ref[...] = acc
        else:
            y_ref[...] = jnp.zeros((m_per, n), jnp.float32)

        out_ref[pl.ds(my * m_per, m_per), :] = y_ref[:, pl.ds(my * n_per, n_per)]
        for s in range(1, N_DEV):
            dst = lax.rem(my + s, N_DEV)
            stage_ref[s] = y_ref[:, pl.ds(dst * n_per, n_per)].astype(jnp.bfloat16)

        if _A2A:
            rdmas = []
            for s in range(1, N_DEV):
                dst = lax.rem(my + s, N_DEV)
                rdma = pltpu.make_async_remote_copy(
                    src_ref=stage_ref.at[s],
                    dst_ref=rstage_ref.at[s],
                    send_sem=send_sems.at[s],
                    recv_sem=recv_sems.at[s],
                    device_id=(dst,),
                    device_id_type=_DeviceIdType.MESH,
                )
                rdma.start()
                rdmas.append(rdma)
            for rdma in rdmas:
                rdma.wait()
            for s in range(1, N_DEV):
                src = lax.rem(my - s + N_DEV, N_DEV)
                out_ref[pl.ds(src * m_per, m_per), :] = (
                    rstage_ref[s].astype(jnp.float32))


    return pl.pallas_call(
        body,
        out_shape=jax.ShapeDtypeStruct((m_out, n_per), jnp.float32),
        in_specs=[
            pl.BlockSpec(memory_space=pltpu.VMEM),
            pl.BlockSpec(memory_space=pltpu.VMEM),
            pl.BlockSpec(memory_space=pltpu.SMEM),
            pl.BlockSpec(memory_space=pltpu.SMEM),
        ],
        out_specs=pl.BlockSpec(memory_space=pltpu.VMEM),
        scratch_shapes=[
            pltpu.VMEM((m_per, n), jnp.float32),
            pltpu.VMEM((N_DEV, m_per, n_per), jnp.bfloat16),
            pltpu.VMEM((N_DEV, m_per, n_per), jnp.bfloat16),
            pltpu.SemaphoreType.DMA((N_DEV,)),
            pltpu.SemaphoreType.DMA((N_DEV,)),
        ],
        compiler_params=pltpu.CompilerParams(
            vmem_limit_bytes=100 * 1024 * 1024,
        ),
    )(x, w_mat, scale_x, scale_w)


# device time: 37696 ns/iter; 1.2902x vs baseline; 1.0266x over previous
import os

import jax
import jax.numpy as jnp
from jax import lax
from jax.experimental import pallas as pl
from jax.experimental.pallas import tpu as pltpu

N_DEV = 16

_CAST = os.environ.get("K_CAST", "fp8")
_A2A = os.environ.get("K_A2A", "1") == "1"
_GEMM = os.environ.get("K_GEMM", "1") == "1"
_NSEND = int(os.environ.get("K_NSEND", str(N_DEV - 1)))

_DeviceIdType = getattr(pl, "DeviceIdType", None) or pltpu.DeviceIdType


def kernel(x, w_mat, scale_x, scale_w):
    m_per, k = x.shape
    _, n = w_mat.shape
    n_per = n // N_DEV
    m_out = m_per * N_DEV

    def body(x_ref, w_ref, sx_ref, sw_ref, out_ref, y_ref, stage_ref,
             rstage_ref, send_sems, recv_sems, bar_sems):
        my = lax.axis_index("i")

        barrier_sem = pltpu.get_barrier_semaphore()
        if os.environ.get("K_BAR", "diss") == "full":
            for s in range(1, N_DEV):
                pl.semaphore_signal(barrier_sem, inc=1,
                                    device_id=(lax.rem(my + s, N_DEV),),
                                    device_id_type=_DeviceIdType.MESH)
            pl.semaphore_wait(barrier_sem, N_DEV - 1)
        else:
            for k in range(4):
                d = 1 << k
                pl.semaphore_signal(barrier_sem, inc=1,
                                    device_id=(lax.rem(my + d, N_DEV),),
                                    device_id_type=_DeviceIdType.MESH)
                pl.semaphore_wait(barrier_sem, 1)

        scale = sx_ref[0] * sw_ref[0]
        if _GEMM:
            xv = x_ref[...]
            wv = w_ref[...]
            if _CAST == "fp8" and xv.dtype == jnp.float32:
                xv = xv.astype(jnp.float8_e4m3fn)
                wv = wv.astype(jnp.float8_e4m3fn)
            elif _CAST == "bf16" and xv.dtype == jnp.float32:
                xv = xv.astype(jnp.bfloat16)
                wv = wv.astype(jnp.bfloat16)
            acc = lax.dot_general(
                xv, wv, (((1,), (0,)), ((), ())),
                preferred_element_type=jnp.float32,
                precision=lax.Precision.DEFAULT,
            ) * scale
            y_ref[...] = acc
        else:
            y_ref[...] = jnp.zeros((m_per, n), jnp.float32)

        out_ref[pl.ds(my * m_per, m_per), :] = y_ref[:, pl.ds(my * n_per, n_per)]
        for s in range(1, N_DEV):
            dst = lax.rem(my + s, N_DEV)
            stage_ref[s] = y_ref[:, pl.ds(dst * n_per, n_per)].astype(jnp.bfloat16)

        if _A2A:
            rdmas = []
            for s in range(1, 1 + _NSEND):
                dst = lax.rem(my + s, N_DEV)
                rdma = pltpu.make_async_remote_copy(
                    src_ref=stage_ref.at[s],
                    dst_ref=rstage_ref.at[s],
                    send_sem=send_sems.at[s],
                    recv_sem=recv_sems.at[s],
                    device_id=(dst,),
                    device_id_type=_DeviceIdType.MESH,
                )
                rdma.start()
                rdmas.append(rdma)
            for rdma in rdmas:
                rdma.wait()
            for s in range(1, 1 + _NSEND):
                src = lax.rem(my - s + N_DEV, N_DEV)
                out_ref[pl.ds(src * m_per, m_per), :] = (
                    rstage_ref[s].astype(jnp.float32))


    return pl.pallas_call(
        body,
        out_shape=jax.ShapeDtypeStruct((m_out, n_per), jnp.float32),
        in_specs=[
            pl.BlockSpec(memory_space=pltpu.VMEM),
            pl.BlockSpec(memory_space=pltpu.VMEM),
            pl.BlockSpec(memory_space=pltpu.SMEM),
            pl.BlockSpec(memory_space=pltpu.SMEM),
        ],
        out_specs=pl.BlockSpec(memory_space=pltpu.VMEM),
        scratch_shapes=[
            pltpu.VMEM((m_per, n), jnp.float32),
            pltpu.VMEM((N_DEV, m_per, n_per), jnp.bfloat16),
            pltpu.VMEM((N_DEV, m_per, n_per), jnp.bfloat16),
            pltpu.SemaphoreType.DMA((N_DEV,)),
            pltpu.SemaphoreType.DMA((N_DEV,)),
            pltpu.SemaphoreType.REGULAR((4,)),
        ],
        compiler_params=pltpu.CompilerParams(
            vmem_limit_bytes=100 * 1024 * 1024,
            collective_id=0,
        ),
    )(x, w_mat, scale_x, scale_w)


# device time: 37638 ns/iter; 1.2922x vs baseline; 1.0015x over previous
import os

import jax
import jax.numpy as jnp
from jax import lax
from jax.experimental import pallas as pl
from jax.experimental.pallas import tpu as pltpu

N_DEV = 16

_CAST = os.environ.get("K_CAST", "fp8")
_A2A = os.environ.get("K_A2A", "1") == "1"
_GEMM = os.environ.get("K_GEMM", "1") == "1"
_NSEND = int(os.environ.get("K_NSEND", str(N_DEV - 1)))
_CHUNK = os.environ.get("K_CHUNK", "1") == "1"

_DeviceIdType = getattr(pl, "DeviceIdType", None) or pltpu.DeviceIdType


def kernel(x, w_mat, scale_x, scale_w):
    m_per, k = x.shape
    _, n = w_mat.shape
    n_per = n // N_DEV
    m_out = m_per * N_DEV

    def body(x_ref, w_ref, sx_ref, sw_ref, out_ref, y_ref, stage_ref,
             rstage_ref, send_sems, recv_sems, bar_sems):
        my = lax.axis_index("i")

        barrier_sem = pltpu.get_barrier_semaphore()
        if os.environ.get("K_BAR", "diss") == "full":
            for s in range(1, N_DEV):
                pl.semaphore_signal(barrier_sem, inc=1,
                                    device_id=(lax.rem(my + s, N_DEV),),
                                    device_id_type=_DeviceIdType.MESH)
            pl.semaphore_wait(barrier_sem, N_DEV - 1)
        else:
            for k in range(4):
                d = 1 << k
                pl.semaphore_signal(barrier_sem, inc=1,
                                    device_id=(lax.rem(my + d, N_DEV),),
                                    device_id_type=_DeviceIdType.MESH)
                pl.semaphore_wait(barrier_sem, 1)

        scale = sx_ref[0] * sw_ref[0]
        if _CHUNK:
            xv = x_ref[...]
            if xv.dtype == jnp.float32 and _CAST == "fp8":
                xv = xv.astype(jnp.float8_e4m3fn)
            rdmas = []
            for s in range(1, N_DEV):
                dst = lax.rem(my + s, N_DEV)
                wv = w_ref[:, pl.ds(dst * n_per, n_per)]
                if wv.dtype == jnp.float32 and _CAST == "fp8":
                    wv = wv.astype(jnp.float8_e4m3fn)
                acc = lax.dot_general(
                    xv, wv, (((1,), (0,)), ((), ())),
                    preferred_element_type=jnp.float32,
                    precision=lax.Precision.DEFAULT,
                ) * scale
                stage_ref[s] = acc.astype(jnp.bfloat16)
                rdma = pltpu.make_async_remote_copy(
                    src_ref=stage_ref.at[s],
                    dst_ref=rstage_ref.at[s],
                    send_sem=send_sems.at[s],
                    recv_sem=recv_sems.at[s],
                    device_id=(dst,),
                    device_id_type=_DeviceIdType.MESH,
                )
                rdma.start()
                rdmas.append(rdma)
            wv = w_ref[:, pl.ds(my * n_per, n_per)]
            if wv.dtype == jnp.float32 and _CAST == "fp8":
                wv = wv.astype(jnp.float8_e4m3fn)
            out_ref[pl.ds(my * m_per, m_per), :] = lax.dot_general(
                xv, wv, (((1,), (0,)), ((), ())),
                preferred_element_type=jnp.float32,
                precision=lax.Precision.DEFAULT,
            ) * scale
            for rdma in rdmas:
                rdma.wait()
            for s in range(1, N_DEV):
                src = lax.rem(my - s + N_DEV, N_DEV)
                out_ref[pl.ds(src * m_per, m_per), :] = (
                    rstage_ref[s].astype(jnp.float32))
            return

        if _GEMM:
            xv = x_ref[...]
            wv = w_ref[...]
            if _CAST == "fp8" and xv.dtype == jnp.float32:
                xv = xv.astype(jnp.float8_e4m3fn)
                wv = wv.astype(jnp.float8_e4m3fn)
            elif _CAST == "bf16" and xv.dtype == jnp.float32:
                xv = xv.astype(jnp.bfloat16)
                wv = wv.astype(jnp.bfloat16)
            acc = lax.dot_general(
                xv, wv, (((1,), (0,)), ((), ())),
                preferred_element_type=jnp.float32,
                precision=lax.Precision.DEFAULT,
            ) * scale
            y_ref[...] = acc
        else:
            y_ref[...] = jnp.zeros((m_per, n), jnp.float32)

        out_ref[pl.ds(my * m_per, m_per), :] = y_ref[:, pl.ds(my * n_per, n_per)]
        for s in range(1, N_DEV):
            dst = lax.rem(my + s, N_DEV)
            stage_ref[s] = y_ref[:, pl.ds(dst * n_per, n_per)].astype(jnp.bfloat16)

        if _A2A:
            rdmas = []
            for s in range(1, 1 + _NSEND):
                dst = lax.rem(my + s, N_DEV)
                rdma = pltpu.make_async_remote_copy(
                    src_ref=stage_ref.at[s],
                    dst_ref=rstage_ref.at[s],
                    send_sem=send_sems.at[s],
                    recv_sem=recv_sems.at[s],
                    device_id=(dst,),
                    device_id_type=_DeviceIdType.MESH,
                )
                rdma.start()
                rdmas.append(rdma)
            for rdma in rdmas:
                rdma.wait()
            for s in range(1, 1 + _NSEND):
                src = lax.rem(my - s + N_DEV, N_DEV)
                out_ref[pl.ds(src * m_per, m_per), :] = (
                    rstage_ref[s].astype(jnp.float32))


    return pl.pallas_call(
        body,
        out_shape=jax.ShapeDtypeStruct((m_out, n_per), jnp.float32),
        in_specs=[
            pl.BlockSpec(memory_space=pltpu.VMEM),
            pl.BlockSpec(memory_space=pltpu.VMEM),
            pl.BlockSpec(memory_space=pltpu.SMEM),
            pl.BlockSpec(memory_space=pltpu.SMEM),
        ],
        out_specs=pl.BlockSpec(memory_space=pltpu.VMEM),
        scratch_shapes=[
            pltpu.VMEM((m_per, n), jnp.float32),
            pltpu.VMEM((N_DEV, m_per, n_per), jnp.bfloat16),
            pltpu.VMEM((N_DEV, m_per, n_per), jnp.bfloat16),
            pltpu.SemaphoreType.DMA((N_DEV,)),
            pltpu.SemaphoreType.DMA((N_DEV,)),
            pltpu.SemaphoreType.REGULAR((4,)),
        ],
        compiler_params=pltpu.CompilerParams(
            vmem_limit_bytes=100 * 1024 * 1024,
            collective_id=0,
        ),
    )(x, w_mat, scale_x, scale_w)


# device time: 30010 ns/iter; 1.6207x vs baseline; 1.2542x over previous
import os

import jax
import jax.numpy as jnp
from jax import lax
from jax.experimental import pallas as pl
from jax.experimental.pallas import tpu as pltpu

N_DEV = 16
N_GRP = 4
_CAST = os.environ.get("K_CAST", "fp8")
_MONO = os.environ.get("K_MONO", "0") == "1"

_DeviceIdType = getattr(pl, "DeviceIdType", None) or pltpu.DeviceIdType


def _mxu(v):
    if v.dtype == jnp.float32 and _CAST == "fp8":
        return v.astype(jnp.float8_e4m3fn)
    if v.dtype == jnp.float32 and _CAST == "bf16":
        return v.astype(jnp.bfloat16)
    return v


def _dot(a, b):
    return lax.dot_general(
        a, b, (((1,), (0,)), ((), ())),
        preferred_element_type=jnp.float32,
        precision=lax.Precision.DEFAULT,
    )


def kernel(x, w_mat, scale_x, scale_w):
    m_per, k = x.shape
    _, n = w_mat.shape
    n_per = n // N_DEV
    m_out = m_per * N_DEV
    d_per_g = N_DEV // N_GRP
    n_g = n // N_GRP

    def body(x_ref, w_ref, sx_ref, sw_ref, out_ref, stage_ref, rstage_ref,
             send_sems, recv_sems):
        my = lax.axis_index("i")

        barrier_sem = pltpu.get_barrier_semaphore()
        for kk in range(4):
            pl.semaphore_signal(barrier_sem, inc=1,
                                device_id=(lax.rem(my + (1 << kk), N_DEV),),
                                device_id_type=_DeviceIdType.MESH)
            pl.semaphore_wait(barrier_sem, 1)

        scale = sx_ref[0] * sw_ref[0]
        xv = _mxu(x_ref[...])

        def send_block(dst, blk, descs):
            @pl.when(dst == my)
            def _():
                out_ref[pl.ds(my * m_per, m_per), :] = blk

            @pl.when(dst != my)
            def _():
                stage_ref[dst] = blk.astype(jnp.bfloat16)

            rdma = pltpu.make_async_remote_copy(
                src_ref=stage_ref.at[dst],
                dst_ref=rstage_ref.at[my],
                send_sem=send_sems.at[dst],
                recv_sem=recv_sems.at[my],
                device_id=(dst,),
                device_id_type=_DeviceIdType.MESH,
            )

            @pl.when(dst != my)
            def _():
                rdma.start()

            descs.append(rdma)

        descs = []
        if _MONO:
            acc = _dot(xv, _mxu(w_ref[...])) * scale
            for dst in range(N_DEV):
                send_block(dst, acc[:, dst * n_per:(dst + 1) * n_per], descs)
        else:
            for g in range(N_GRP):
                wv = _mxu(w_ref[:, g * n_g:(g + 1) * n_g])
                acc = _dot(xv, wv) * scale
                for j in range(d_per_g):
                    dst = g * d_per_g + j
                    send_block(dst, acc[:, j * n_per:(j + 1) * n_per], descs)

        for src in range(N_DEV):
            recv = pltpu.make_async_remote_copy(
                src_ref=stage_ref.at[src],
                dst_ref=rstage_ref.at[src],
                send_sem=send_sems.at[src],
                recv_sem=recv_sems.at[src],
                device_id=(src,),
                device_id_type=_DeviceIdType.MESH,
            )

            @pl.when(src != my)
            def _():
                recv.wait_recv()
                out_ref[pl.ds(src * m_per, m_per), :] = (
                    rstage_ref[src].astype(jnp.float32))

        for dst in range(N_DEV):
            @pl.when(dst != my)
            def _():
                descs[dst].wait_send()

    return pl.pallas_call(
        body,
        out_shape=jax.ShapeDtypeStruct((m_out, n_per), jnp.float32),
        in_specs=[
            pl.BlockSpec(memory_space=pltpu.VMEM),
            pl.BlockSpec(memory_space=pltpu.VMEM),
            pl.BlockSpec(memory_space=pltpu.SMEM),
            pl.BlockSpec(memory_space=pltpu.SMEM),
        ],
        out_specs=pl.BlockSpec(memory_space=pltpu.VMEM),
        scratch_shapes=[
            pltpu.VMEM((N_DEV, m_per, n_per), jnp.bfloat16),
            pltpu.VMEM((N_DEV, m_per, n_per), jnp.bfloat16),
            pltpu.SemaphoreType.DMA((N_DEV,)),
            pltpu.SemaphoreType.DMA((N_DEV,)),
        ],
        compiler_params=pltpu.CompilerParams(
            vmem_limit_bytes=100 * 1024 * 1024,
            collective_id=0,
        ),
    )(x, w_mat, scale_x, scale_w)


# device time: 27058 ns/iter; 1.7975x vs baseline; 1.1091x over previous
import os

import jax
import jax.numpy as jnp
from jax import lax
from jax.experimental import pallas as pl
from jax.experimental.pallas import tpu as pltpu

N_DEV = 16
N_GRP = 4
_CAST = os.environ.get("K_CAST", "fp8")
_MONO = os.environ.get("K_MONO", "0") == "1"

_DeviceIdType = getattr(pl, "DeviceIdType", None) or pltpu.DeviceIdType


def _mxu(v):
    if v.dtype == jnp.float32 and _CAST == "fp8":
        return v.astype(jnp.float8_e4m3fn)
    if v.dtype == jnp.float32 and _CAST == "bf16":
        return v.astype(jnp.bfloat16)
    return v


def _dot(a, b):
    return lax.dot_general(
        a, b, (((1,), (0,)), ((), ())),
        preferred_element_type=jnp.float32,
        precision=lax.Precision.DEFAULT,
    )


def kernel(x, w_mat, scale_x, scale_w):
    m_per, k = x.shape
    _, n = w_mat.shape
    n_per = n // N_DEV
    m_out = m_per * N_DEV
    d_per_g = N_DEV // N_GRP
    n_g = n // N_GRP

    def body(x_ref, w_ref, sx_ref, sw_ref, out_ref, wbuf_ref, stage_ref,
             rstage_ref, send_sems, recv_sems, load_sems):
        my = lax.axis_index("i")

        loads = []

        def start_load(g):
            cp = pltpu.make_async_copy(
                w_ref.at[:, g * n_g:(g + 1) * n_g],
                wbuf_ref.at[g % 2],
                load_sems.at[g % 2],
            )
            cp.start()
            loads.append(cp)

        start_load(0)

        barrier_sem = pltpu.get_barrier_semaphore()
        for kk in range(4):
            pl.semaphore_signal(barrier_sem, inc=1,
                                device_id=(lax.rem(my + (1 << kk), N_DEV),),
                                device_id_type=_DeviceIdType.MESH)
            pl.semaphore_wait(barrier_sem, 1)

        scale = sx_ref[0] * sw_ref[0]
        xv = _mxu(x_ref[...])

        def send_block(dst, blk, descs):
            @pl.when(dst == my)
            def _():
                out_ref[pl.ds(my * m_per, m_per), :] = blk

            @pl.when(dst != my)
            def _():
                stage_ref[dst] = blk.astype(jnp.bfloat16)

            rdma = pltpu.make_async_remote_copy(
                src_ref=stage_ref.at[dst],
                dst_ref=rstage_ref.at[my],
                send_sem=send_sems.at[dst],
                recv_sem=recv_sems.at[my],
                device_id=(dst,),
                device_id_type=_DeviceIdType.MESH,
            )

            @pl.when(dst != my)
            def _():
                rdma.start()

            descs.append((dst, rdma))

        descs = []
        for g in range(N_GRP):
            if g + 1 < N_GRP:
                start_load(g + 1)
            loads[g].wait()
            wv = _mxu(wbuf_ref[g % 2])
            acc = _dot(xv, wv) * scale
            for j in range(d_per_g):
                dst = g * d_per_g + j
                send_block(dst, acc[:, j * n_per:(j + 1) * n_per], descs)

        for src in range(N_DEV):
            recv = pltpu.make_async_remote_copy(
                src_ref=stage_ref.at[src],
                dst_ref=rstage_ref.at[src],
                send_sem=send_sems.at[src],
                recv_sem=recv_sems.at[src],
                device_id=(src,),
                device_id_type=_DeviceIdType.MESH,
            )

            @pl.when(src != my)
            def _():
                recv.wait_recv()
                out_ref[pl.ds(src * m_per, m_per), :] = (
                    rstage_ref[src].astype(jnp.float32))

        for dst, rdma in descs:
            @pl.when(dst != my)
            def _():
                rdma.wait_send()

    return pl.pallas_call(
        body,
        out_shape=jax.ShapeDtypeStruct((m_out, n_per), jnp.float32),
        in_specs=[
            pl.BlockSpec(memory_space=pltpu.VMEM),
            pl.BlockSpec(memory_space=pl.ANY),
            pl.BlockSpec(memory_space=pltpu.SMEM),
            pl.BlockSpec(memory_space=pltpu.SMEM),
        ],
        out_specs=pl.BlockSpec(memory_space=pltpu.VMEM),
        scratch_shapes=[
            pltpu.VMEM((2, k, n_g), jnp.float32),
            pltpu.VMEM((N_DEV, m_per, n_per), jnp.bfloat16),
            pltpu.VMEM((N_DEV, m_per, n_per), jnp.bfloat16),
            pltpu.SemaphoreType.DMA((N_DEV,)),
            pltpu.SemaphoreType.DMA((N_DEV,)),
            pltpu.SemaphoreType.DMA((2,)),
        ],
        compiler_params=pltpu.CompilerParams(
            vmem_limit_bytes=100 * 1024 * 1024,
            collective_id=0,
        ),
    )(x, w_mat, scale_x, scale_w)
